# baseline (speedup 1.0000x reference)
import functools

import jax
import jax.numpy as jnp
from jax import lax
from jax.experimental import pallas as pl
from jax.experimental.pallas import tpu as pltpu
from jax.experimental.pallas import tpu_sc as plsc

_VOCAB = 100000
_EMBED = 64
_HIDDEN = 128
_CTX = 200

_NC = 2
_NS = 16
_NW = _NC * _NS
_IDX_PER_W = 8
_ACTIVE_W = _CTX // _IDX_PER_W

_TILE = 16384
_V_SC = 32768
_SC_OFF_BLKS = _V_SC // _TILE
_ROWS_PER_W = _V_SC // _NW
_CHUNK = 256
_V_TC = _VOCAB - _V_SC
_GRID = (_V_TC + _TILE - 1) // _TILE


def _sc_gather_sum(idx_hbm, embt_hbm, out_hbm, idx_v, rows_v, acc_v, sem):
    wid = lax.axis_index("s") * _NC + lax.axis_index("c")

    @pl.when(wid < _ACTIVE_W)
    def _gather():
        base = pl.multiple_of(wid * _IDX_PER_W, _IDX_PER_W)
        pltpu.sync_copy(idx_hbm.at[pl.ds(base, _IDX_PER_W)],
                        idx_v.at[pl.ds(0, _IDX_PER_W)])
        iv = idx_v[...]
        lane = lax.iota(jnp.int32, 16)
        copies = []
        cols = []
        for j in range(_IDX_PER_W):
            col_j = jnp.sum(jnp.where(lane == j, iv, 0))
            start = pl.multiple_of((col_j >> 7) << 7, 128)
            cols.append(col_j & 127)
            copies.append(pltpu.make_async_copy(
                embt_hbm.at[:, pl.ds(start, 128)], rows_v.at[j], sem))
        for c in copies:
            c.start()
        for c in copies:
            c.wait()
        for c in range(_EMBED // 16):
            rid = lax.iota(jnp.int32, 16) + c * 16
            acc = jnp.zeros((16,), jnp.float32)
            for j in range(_IDX_PER_W):
                cj = jnp.broadcast_to(cols[j], (16,))
                acc = acc + plsc.load_gather(rows_v.at[j], [rid, cj])
            acc_v[0, pl.ds(c * 16, 16)] = acc

    @pl.when(wid >= _ACTIVE_W)
    def _zero():
        for c in range(_EMBED // 16):
            acc_v[0, pl.ds(c * 16, 16)] = jnp.zeros((16,), jnp.float32)

    pltpu.sync_copy(acc_v, out_hbm.at[pl.ds(wid, 1)])


def _sc_matvec(hid_hbm, w2_hbm, out_hbm, hid_v, buf_v, acc_v, sem_a, sem_b):
    wid = lax.axis_index("s") * _NC + lax.axis_index("c")
    base = pl.multiple_of(wid * _ROWS_PER_W, _ROWS_PER_W)
    pltpu.sync_copy(hid_hbm, hid_v)
    hid_c = [hid_v[0, pl.ds(c * 16, 16)] for c in range(_HIDDEN // 16)]
    nchunk = _ROWS_PER_W // _CHUNK
    sems = [sem_a, sem_b]
    copies = [
        pltpu.make_async_copy(
            w2_hbm.at[pl.ds(base + k * _CHUNK, _CHUNK)],
            buf_v.at[k % 2], sems[k % 2])
        for k in range(nchunk)
    ]
    copies[0].start()
    for k in range(nchunk):
        if k + 1 < nchunk:
            copies[k + 1].start()
        copies[k].wait()

        lane = lax.iota(jnp.int32, 16)

        def row_body(g, _, k=k):
            res = jnp.zeros((16,), jnp.float32)
            for u in range(16):
                r = g * 16 + u
                ts = []
                for c in range(_HIDDEN // 16):
                    ts.append(buf_v[k % 2, r, pl.ds(c * 16, 16)] * hid_c[c])
                while len(ts) > 1:
                    ts = [a + b for a, b in zip(ts[0::2], ts[1::2])]
                res = jnp.where(lane == u, jnp.sum(ts[0]), res)
            acc_v[0, pl.ds(k * _CHUNK + g * 16, 16)] = res
            return 0

        lax.fori_loop(0, _CHUNK // 16, row_body, 0)
    pltpu.sync_copy(acc_v, out_hbm.at[:, pl.ds(wid * _ROWS_PER_W, _ROWS_PER_W)])


@functools.cache
def _sc_calls():
    gather = pl.kernel(
        _sc_gather_sum,
        out_type=jax.ShapeDtypeStruct((_NW, _EMBED), jnp.float32),
        mesh=plsc.VectorSubcoreMesh(core_axis_name="c", subcore_axis_name="s"),
        scratch_types=[
            pltpu.VMEM((16,), jnp.int32),
            pltpu.VMEM((_IDX_PER_W, _EMBED, 128), jnp.float32),
            pltpu.VMEM((1, _EMBED), jnp.float32),
            pltpu.SemaphoreType.DMA,
        ],
        compiler_params=pltpu.CompilerParams(needs_layout_passes=False),
    )
    matvec = pl.kernel(
        _sc_matvec,
        out_type=jax.ShapeDtypeStruct((1, _V_SC), jnp.float32),
        mesh=plsc.VectorSubcoreMesh(core_axis_name="c", subcore_axis_name="s"),
        scratch_types=[
            pltpu.VMEM((1, _HIDDEN), jnp.float32),
            pltpu.VMEM((2, _CHUNK, _HIDDEN), jnp.float32),
            pltpu.VMEM((1, _ROWS_PER_W), jnp.float32),
            pltpu.SemaphoreType.DMA,
            pltpu.SemaphoreType.DMA,
        ],
        compiler_params=pltpu.CompilerParams(needs_layout_passes=False),
    )
    return gather, matvec


def _tc_hidden(parts_ref, w1_ref, b1_ref, out_ref):
    x = jnp.sum(parts_ref[...], axis=0, keepdims=True)
    h = lax.dot_general(
        x, w1_ref[...], (((1,), (1,)), ((), ())),
        preferred_element_type=jnp.float32,
    ) + b1_ref[...]
    out_ref[...] = jnp.maximum(h, 0.0)


_tc_hidden_call = pl.pallas_call(
    _tc_hidden,
    in_specs=[
        pl.BlockSpec((_NW, _EMBED), lambda: (0, 0)),
        pl.BlockSpec((_HIDDEN, _EMBED), lambda: (0, 0)),
        pl.BlockSpec((1, _HIDDEN), lambda: (0, 0)),
    ],
    out_specs=pl.BlockSpec((1, _HIDDEN), lambda: (0, 0)),
    out_shape=jax.ShapeDtypeStruct((1, _HIDDEN), jnp.float32),
)


def _tc_logits(hid_ref, w2_ref, b2_ref, out_ref):
    out_ref[...] = lax.dot_general(
        hid_ref[...], w2_ref[...], (((1,), (1,)), ((), ())),
        preferred_element_type=jnp.float32,
    ) + b2_ref[...].reshape(1, _TILE)


_tc_logits_call = pl.pallas_call(
    _tc_logits,
    grid=(_GRID,),
    in_specs=[
        pl.BlockSpec((1, _HIDDEN), lambda i: (0, 0)),
        pl.BlockSpec((_TILE, _HIDDEN), lambda i: (i + _SC_OFF_BLKS, 0)),
        pl.BlockSpec((_TILE,), lambda i: (i + _SC_OFF_BLKS,)),
    ],
    out_specs=pl.BlockSpec((1, _TILE), lambda i: (0, i)),
    out_shape=jax.ShapeDtypeStruct((1, _V_TC), jnp.float32),
    compiler_params=pltpu.CompilerParams(
        dimension_semantics=("arbitrary",),
    ),
)


def _tc_logsoftmax(sc_ref, b2sc_ref, tc_ref, out_ref):
    a = sc_ref[...] + b2sc_ref[...].reshape(1, _V_SC)
    b = tc_ref[...]
    m = jnp.maximum(jnp.max(a), jnp.max(b))
    lse = m + jnp.log(jnp.sum(jnp.exp(a - m)) + jnp.sum(jnp.exp(b - m)))
    out_ref[:, pl.ds(0, _V_SC)] = a - lse
    out_ref[:, pl.ds(_V_SC, _V_TC)] = b - lse


_tc_norm_call = pl.pallas_call(
    _tc_logsoftmax,
    grid=(1,),
    in_specs=[
        pl.BlockSpec((1, _V_SC), lambda i: (0, 0)),
        pl.BlockSpec((_V_SC,), lambda i: (0,)),
        pl.BlockSpec((1, _V_TC), lambda i: (0, 0)),
    ],
    out_specs=pl.BlockSpec((1, _VOCAB), lambda i: (0, 0)),
    out_shape=jax.ShapeDtypeStruct((1, _VOCAB), jnp.float32),
)


@jax.jit
def kernel(inputs, emb, W1, b1, W2, b2):
    gather, matvec = _sc_calls()
    parts = gather(inputs, emb.T)
    hidden = _tc_hidden_call(parts, W1, b1.reshape(1, _HIDDEN))
    lg_sc = matvec(hidden, W2)
    lg_tc = _tc_logits_call(hidden, W2, b2)
    return _tc_norm_call(lg_sc, b2, lg_tc)


# speedup vs baseline: 1.1033x; 1.1033x over previous
import functools

import jax
import jax.numpy as jnp
from jax import lax
from jax.experimental import pallas as pl
from jax.experimental.pallas import tpu as pltpu
from jax.experimental.pallas import tpu_sc as plsc

_VOCAB = 100000
_EMBED = 64
_HIDDEN = 128
_CTX = 200

_NC = 2
_NS = 16
_NW = _NC * _NS
_IDX_PER_W = 8
_ACTIVE_W = _CTX // _IDX_PER_W

_TILE = 16384
_GRID = (_VOCAB + _TILE - 1) // _TILE
_EDGE = _VOCAB - (_GRID - 1) * _TILE


def _sc_gather_sum(idx_hbm, embt_hbm, out_hbm, idx_v, rows_v, acc_v, sem):
    wid = lax.axis_index("s") * _NC + lax.axis_index("c")

    @pl.when(wid < _ACTIVE_W)
    def _gather():
        base = pl.multiple_of(wid * _IDX_PER_W, _IDX_PER_W)
        pltpu.sync_copy(idx_hbm.at[pl.ds(base, _IDX_PER_W)],
                        idx_v.at[pl.ds(0, _IDX_PER_W)])
        iv = idx_v[...]
        lane = lax.iota(jnp.int32, 16)
        copies = []
        cols = []
        for j in range(_IDX_PER_W):
            col_j = jnp.sum(jnp.where(lane == j, iv, 0))
            start = pl.multiple_of((col_j >> 7) << 7, 128)
            cols.append(col_j & 127)
            copies.append(pltpu.make_async_copy(
                embt_hbm.at[:, pl.ds(start, 128)], rows_v.at[j], sem))
        for c in copies:
            c.start()
        for c in copies:
            c.wait()
        for c in range(_EMBED // 16):
            rid = lax.iota(jnp.int32, 16) + c * 16
            acc = jnp.zeros((16,), jnp.float32)
            for j in range(_IDX_PER_W):
                cj = jnp.broadcast_to(cols[j], (16,))
                acc = acc + plsc.load_gather(rows_v.at[j], [rid, cj])
            acc_v[0, pl.ds(c * 16, 16)] = acc

    @pl.when(wid >= _ACTIVE_W)
    def _zero():
        for c in range(_EMBED // 16):
            acc_v[0, pl.ds(c * 16, 16)] = jnp.zeros((16,), jnp.float32)

    pltpu.sync_copy(acc_v, out_hbm.at[pl.ds(wid, 1)])


@functools.cache
def _sc_gather():
    return pl.kernel(
        _sc_gather_sum,
        out_type=jax.ShapeDtypeStruct((_NW, _EMBED), jnp.float32),
        mesh=plsc.VectorSubcoreMesh(core_axis_name="c", subcore_axis_name="s"),
        scratch_types=[
            pltpu.VMEM((16,), jnp.int32),
            pltpu.VMEM((_IDX_PER_W, _EMBED, 128), jnp.float32),
            pltpu.VMEM((1, _EMBED), jnp.float32),
            pltpu.SemaphoreType.DMA,
        ],
        compiler_params=pltpu.CompilerParams(needs_layout_passes=False),
    )


def _tc_logits(parts_ref, w1_ref, b1_ref, w2_ref, b2_ref, out_ref, hid_ref):
    i = pl.program_id(0)

    @pl.when(i == 0)
    def _head():
        x = jnp.sum(parts_ref[...], axis=0, keepdims=True)
        h = lax.dot_general(
            x, w1_ref[...], (((1,), (1,)), ((), ())),
            preferred_element_type=jnp.float32,
        ) + b1_ref[...]
        hid_ref[...] = jnp.maximum(h, 0.0)

    out_ref[...] = lax.dot_general(
        hid_ref[...], w2_ref[...], (((1,), (1,)), ((), ())),
        preferred_element_type=jnp.float32,
    ) + b2_ref[...].reshape(1, _TILE)


_tc_logits_call = pl.pallas_call(
    _tc_logits,
    grid=(_GRID,),
    in_specs=[
        pl.BlockSpec((_NW, _EMBED), lambda i: (0, 0)),
        pl.BlockSpec((_HIDDEN, _EMBED), lambda i: (0, 0)),
        pl.BlockSpec((1, _HIDDEN), lambda i: (0, 0)),
        pl.BlockSpec((_TILE, _HIDDEN), lambda i: (i, 0)),
        pl.BlockSpec((_TILE,), lambda i: (i,)),
    ],
    out_specs=pl.BlockSpec((1, _TILE), lambda i: (0, i)),
    out_shape=jax.ShapeDtypeStruct((1, _VOCAB), jnp.float32),
    scratch_shapes=[pltpu.VMEM((1, _HIDDEN), jnp.float32)],
    compiler_params=pltpu.CompilerParams(
        dimension_semantics=("arbitrary",),
    ),
)


def _tc_logsoftmax(lg_ref, out_ref):
    full = lg_ref[...]
    m = jnp.max(full)
    lse = m + jnp.log(jnp.sum(jnp.exp(full - m)))
    out_ref[...] = full - lse


_tc_norm_call = pl.pallas_call(
    _tc_logsoftmax,
    in_specs=[pl.BlockSpec((1, _VOCAB), lambda: (0, 0))],
    out_specs=pl.BlockSpec((1, _VOCAB), lambda: (0, 0)),
    out_shape=jax.ShapeDtypeStruct((1, _VOCAB), jnp.float32),
)


@jax.jit
def kernel(inputs, emb, W1, b1, W2, b2):
    parts = _sc_gather()(inputs, emb.T)
    logits = _tc_logits_call(parts, W1, b1.reshape(1, _HIDDEN), W2, b2)
    return _tc_norm_call(logits)
